# baseline (speedup 1.0000x reference)
import functools

import jax
import jax.numpy as jnp
from jax import lax
from jax.experimental import pallas as pl
from jax.experimental.pallas import tpu as pltpu
from jax.experimental.pallas import tpu_sc as plsc

B = 4096
IN = 128
H = 128
R = 8
MPHY = 4
K = 1024
D = 256
TAU = 1.0
BPS = 2
KBITS = 10
TEMP_MOD = 0.5

TB = 256
NBLK = B // TB
TB2 = 256
NB2 = B // TB2 + R
NROWS = NB2 * TB2
TH = 64
KC = 128
XSW = 128


def _router_kernel(phi_ref, ln_g_ref, ln_b_ref, W1_ref, b1_ref,
                   W2_ref, b2_ref, We_ref, be_ref, Wm_ref, bm_ref,
                   jp_ref, mi_ref, gate_ref, ex_ref, pos_ref, cnt_ref,
                   run_scr):
    @pl.when(pl.program_id(0) == 0)
    def _():
        run_scr[...] = jnp.zeros((1, R), jnp.float32)

    phi = phi_ref[...]
    mu = jnp.mean(phi, axis=-1, keepdims=True)
    var = jnp.mean((phi - mu) ** 2, axis=-1, keepdims=True)
    phin = (phi - mu) / jnp.sqrt(var + 1e-5) * ln_g_ref[...] + ln_b_ref[...]
    h = jax.nn.gelu(jnp.dot(phin, W1_ref[...]) + b1_ref[...])
    h = jax.nn.gelu(jnp.dot(h, W2_ref[...]) + b2_ref[...])
    logits_e = jnp.dot(h, We_ref[...]) + be_ref[...]
    logits_m = jnp.dot(h, Wm_ref[...]) + bm_ref[...]
    p_e = jax.nn.softmax(logits_e / TAU, axis=-1)
    p_m = jax.nn.softmax(logits_m / TAU, axis=-1)
    for e in range(R):
        jp_ref[:, MPHY * e:MPHY * (e + 1)] = p_e[:, e:e + 1] * p_m
    jl_max = (jnp.max(logits_e, axis=-1, keepdims=True)
              + jnp.max(logits_m, axis=-1, keepdims=True))
    iota_m = lax.broadcasted_iota(jnp.int32, (TB, MPHY), 1)
    mi = jnp.full((TB, 1), R * MPHY, jnp.int32)
    for e in range(R):
        eq = (logits_e[:, e:e + 1] + logits_m) == jl_max
        cand = jnp.min(jnp.where(eq, iota_m + MPHY * e, R * MPHY), axis=-1,
                       keepdims=True)
        mi = jnp.minimum(mi, cand)
    iota_e8 = lax.broadcasted_iota(jnp.int32, (TB, R), 1)
    expert = mi // MPHY
    pe_sel = jnp.sum(jnp.where(iota_e8 == expert, p_e, 0.0), axis=-1,
                     keepdims=True)
    pm_sel = jnp.sum(jnp.where(iota_m == mi - MPHY * expert, p_m, 0.0),
                     axis=-1, keepdims=True)
    gate = pe_sel * pm_sel
    mi_ref[...] = mi
    gate_ref[...] = gate
    ex_ref[...] = expert

    iota_e = lax.broadcasted_iota(jnp.int32, (TB, R), 1)
    oh = (expert == iota_e)
    oh_f = oh.astype(jnp.float32)
    r_i = lax.broadcasted_iota(jnp.int32, (TB, TB), 0)
    c_i = lax.broadcasted_iota(jnp.int32, (TB, TB), 1)
    ltri = (c_i <= r_i).astype(jnp.float32)
    ranks = jnp.dot(ltri, oh_f)
    cnt_blk = ranks[TB - 1:TB, :]
    run_row = run_scr[...]
    pick_rank = jnp.sum(jnp.where(oh, ranks, 0.0), axis=1, keepdims=True)
    pick_run = jnp.sum(jnp.where(oh, run_row, 0.0), axis=1, keepdims=True)
    pos = pick_run + pick_rank - 1.0
    pos_ref[...] = pos.astype(jnp.int32)
    new_run = run_row + cnt_blk
    run_scr[...] = new_run
    cnt_ref[...] = new_run.astype(jnp.int32)


def _router(phi, ln_g, ln_b, W1, b1, W2, b2, We, be, Wm, bm):
    const_spec = lambda shape: pl.BlockSpec(shape, lambda i: (0, 0))
    tok_spec = lambda shape: pl.BlockSpec(shape, lambda i: (i, 0))
    return pl.pallas_call(
        _router_kernel,
        grid=(NBLK,),
        in_specs=[
            tok_spec((TB, IN)),
            const_spec((1, IN)),
            const_spec((1, IN)),
            const_spec((IN, H)),
            const_spec((1, H)),
            const_spec((H, H)),
            const_spec((1, H)),
            const_spec((H, R)),
            const_spec((1, R)),
            const_spec((H, MPHY)),
            const_spec((1, MPHY)),
        ],
        out_specs=[
            tok_spec((TB, R * MPHY)),
            tok_spec((TB, 1)),
            tok_spec((TB, 1)),
            tok_spec((TB, 1)),
            tok_spec((TB, 1)),
            pl.BlockSpec((1, R), lambda i: (0, 0)),
        ],
        out_shape=[
            jax.ShapeDtypeStruct((B, R * MPHY), jnp.float32),
            jax.ShapeDtypeStruct((B, 1), jnp.int32),
            jax.ShapeDtypeStruct((B, 1), jnp.float32),
            jax.ShapeDtypeStruct((B, 1), jnp.int32),
            jax.ShapeDtypeStruct((B, 1), jnp.int32),
            jax.ShapeDtypeStruct((1, R), jnp.int32),
        ],
        scratch_shapes=[pltpu.VMEM((1, R), jnp.float32)],
    )(phi, ln_g.reshape(1, IN), ln_b.reshape(1, IN), W1, b1.reshape(1, H),
      W2, b2.reshape(1, H), We, be.reshape(1, R), Wm, bm.reshape(1, MPHY))


def _norms_kernel(cb_ref, nc_ref):
    for e in range(R):
        cbe = cb_ref[e * K:(e + 1) * K, :]
        nc_ref[0:1, e * K:(e + 1) * K] = jnp.sum(cbe * cbe, axis=-1)[None, :]


def _norms(cb_flat):
    return pl.pallas_call(
        _norms_kernel,
        out_shape=jax.ShapeDtypeStruct((1, R * K), jnp.float32),
    )(cb_flat)


def _vq_kernel(eb_ref, z_ref, cb_ref, nc_ref, T_ref, zq_ref, xs_ref):
    z = z_ref[...]
    rn = jnp.sum(z * z, axis=-1, keepdims=True)
    cbe = cb_ref[...].reshape(K, D)
    nce = nc_ref[...].reshape(1, K)
    mm = lax.dot_general(z, cbe, (((1,), (1,)), ((), ())))
    lane = lax.broadcasted_iota(jnp.int32, (TH, KC), 1)
    nh = TB2 // TH
    outs = []
    for hh in range(nh):
        sl = slice(hh * TH, (hh + 1) * TH)
        rnh = rn[sl]
        rmin = None
        for c in range(K // KC):
            mmc = mm[sl, c * KC:(c + 1) * KC]
            ncc = nce[:, c * KC:(c + 1) * KC]
            dm = (rnh - 2.0 * mmc) + ncc
            if c == 0:
                rmin = dm
                ridx = jnp.zeros((TH, KC), jnp.int32)
            else:
                upd = dm < rmin
                ridx = jnp.where(upd, c, ridx)
                rmin = jnp.where(upd, dm, rmin)
        tmin = jnp.min(rmin, axis=-1, keepdims=True)
        fidx = ridx * KC + lane
        isel = jnp.min(jnp.where(rmin == tmin, fidx, K), axis=-1,
                       keepdims=True)
        outs.append(isel)
    code = jnp.concatenate(outs, axis=0)

    iota_full = lax.broadcasted_iota(jnp.int32, (TB2, K), 1)
    ohq = (iota_full == code).astype(jnp.float32)
    zq_ref[...] = jnp.dot(ohq, cbe)

    cols = []
    for s in range(KBITS // BPS):
        pr = jnp.bitwise_and(
            lax.shift_right_logical(code, KBITS - BPS - BPS * s), 3)
        for c in range(2):
            v = jnp.where(pr == 0, T_ref[0, c],
                jnp.where(pr == 1, T_ref[1, c],
                jnp.where(pr == 2, T_ref[2, c], T_ref[3, c])))
            cols.append(v)
    cols.append(jnp.zeros((TB2, XSW - 2 * (KBITS // BPS)), jnp.float32))
    xs_ref[...] = jnp.concatenate(cols, axis=1)


def _vq(e_of_blk, z_disp, cb3, nc3, T):
    grid_spec = pltpu.PrefetchScalarGridSpec(
        num_scalar_prefetch=1,
        grid=(NB2,),
        in_specs=[
            pl.BlockSpec((TB2, D), lambda i, eb: (i, 0)),
            pl.BlockSpec((1, K, D), lambda i, eb: (eb[i], 0, 0)),
            pl.BlockSpec((1, 1, K), lambda i, eb: (eb[i], 0, 0)),
            pl.BlockSpec((4, 2), lambda i, eb: (0, 0)),
        ],
        out_specs=[
            pl.BlockSpec((TB2, D), lambda i, eb: (i, 0)),
            pl.BlockSpec((TB2, XSW), lambda i, eb: (i, 0)),
        ],
    )
    return pl.pallas_call(
        _vq_kernel,
        grid_spec=grid_spec,
        out_shape=[
            jax.ShapeDtypeStruct((NROWS, D), jnp.float32),
            jax.ShapeDtypeStruct((NROWS, XSW), jnp.float32),
        ],
    )(e_of_blk, z_disp, cb3, nc3, T)


def _sc_info():
    info = plsc.get_sparse_core_info()
    return info.num_cores, info.num_subcores


def _sc_dispatch(z, slot):
    nc, ns = _sc_info()
    nw = nc * ns
    bpw = B // nw
    mesh = plsc.VectorSubcoreMesh(core_axis_name="c", subcore_axis_name="s")

    @functools.partial(
        pl.kernel,
        out_type=jax.ShapeDtypeStruct((NROWS, D), jnp.float32),
        mesh=mesh,
        scratch_types=[
            pltpu.VMEM((bpw,), jnp.int32),
            pltpu.VMEM((bpw, D), jnp.float32),
            pltpu.SemaphoreType.DMA,
        ],
    )
    def disp_k(z_hbm, slot_hbm, out_hbm, slot_v, rows_v, sem):
        wid = lax.axis_index("s") * nc + lax.axis_index("c")
        base = wid * bpw
        pltpu.sync_copy(slot_hbm.at[pl.ds(base, bpw)], slot_v)
        pltpu.sync_copy(z_hbm.at[pl.ds(base, bpw)], rows_v)
        pltpu.async_copy(rows_v, out_hbm.at[slot_v], sem).wait()

    return disp_k(z, slot)


def _sc_undispatch(zq_disp, xsd, slot):
    nc, ns = _sc_info()
    nw = nc * ns
    bpw = B // nw
    mesh = plsc.VectorSubcoreMesh(core_axis_name="c", subcore_axis_name="s")

    @functools.partial(
        pl.kernel,
        out_type=[
            jax.ShapeDtypeStruct((B, D), jnp.float32),
            jax.ShapeDtypeStruct((B, XSW), jnp.float32),
        ],
        mesh=mesh,
        scratch_types=[
            pltpu.VMEM((bpw,), jnp.int32),
            pltpu.VMEM((bpw, D), jnp.float32),
            pltpu.VMEM((bpw, XSW), jnp.float32),
            pltpu.SemaphoreType.DMA,
            pltpu.SemaphoreType.DMA,
        ],
    )
    def und_k(zqd_hbm, xsd_hbm, slot_hbm, zq_hbm, xss_hbm,
              slot_v, rows_v, xsr_v, sem, sem2):
        wid = lax.axis_index("s") * nc + lax.axis_index("c")
        base = wid * bpw
        pltpu.sync_copy(slot_hbm.at[pl.ds(base, bpw)], slot_v)
        a = pltpu.async_copy(zqd_hbm.at[slot_v], rows_v, sem)
        b = pltpu.async_copy(xsd_hbm.at[slot_v], xsr_v, sem2)
        a.wait()
        pltpu.sync_copy(rows_v, zq_hbm.at[pl.ds(base, bpw)])
        b.wait()
        pltpu.sync_copy(xsr_v, xss_hbm.at[pl.ds(base, bpw)])

    return und_k(zq_disp, xsd, slot)


def _combine_kernel(z_ref, zq_ref, xss_ref, gate_ref, out_ref, xs_ref):
    z = z_ref[...]
    zq = zq_ref[...]
    out_ref[...] = (z + (zq - z)) * gate_ref[...]
    xs_ref[...] = xss_ref[:, 0:2 * (KBITS // BPS)]


def _combine(z, zq_s, xs_s, gate):
    tok_spec = lambda shape: pl.BlockSpec(shape, lambda i: (i, 0))
    return pl.pallas_call(
        _combine_kernel,
        grid=(NBLK,),
        in_specs=[
            tok_spec((TB, D)),
            tok_spec((TB, D)),
            tok_spec((TB, XSW)),
            tok_spec((TB, 1)),
        ],
        out_specs=[tok_spec((TB, D)), tok_spec((TB, 2 * (KBITS // BPS)))],
        out_shape=[
            jax.ShapeDtypeStruct((B, D), jnp.float32),
            jax.ShapeDtypeStruct((B, 2 * (KBITS // BPS)), jnp.float32),
        ],
    )(z, zq_s, xs_s, gate)


def _int_to_bits(x, num_bits):
    shifts = jnp.arange(num_bits - 1, -1, -1)
    return ((x[..., None] >> shifts) & 1).astype(jnp.float32)


def _qam_table():
    import numpy as np
    m_side = int(np.sqrt(1 << BPS))
    levels = jnp.arange(-(m_side - 1), m_side + 1, 2).astype(jnp.float32)
    xs, ys = jnp.meshgrid(levels, levels, indexing='ij')
    pts = jnp.stack([xs.reshape(-1), ys.reshape(-1)], axis=-1)
    max_power = (pts ** 2).sum(axis=-1).max()
    const = pts / jnp.sqrt(max_power + 1e-9)
    cand_bits = _int_to_bits(jnp.arange(1 << BPS), BPS)
    patt = cand_bits
    d_bits = ((patt[:, None, :] - cand_bits[None, :, :]) ** 2).sum(axis=-1)
    w_sym = jax.nn.softmax(-d_bits / max(TEMP_MOD, 1e-6), axis=1)
    return w_sym @ const


def kernel(z, phi, ln_g, ln_b, W1, b1, W2, b2, We, be, Wm, bm, codebooks):
    cb_flat = codebooks.reshape(R * K, D)
    T = _qam_table()

    jp, mi, gate, ex2, pos2, cnt2 = _router(phi, ln_g, ln_b, W1, b1, W2, b2,
                                            We, be, Wm, bm)

    counts = cnt2.reshape(R)
    nblk_e = (counts + TB2 - 1) // TB2
    cum_nblk = jnp.cumsum(nblk_e)
    blk_off = cum_nblk - nblk_e
    e_of_blk = jnp.minimum(
        jnp.sum((jnp.arange(NB2)[:, None] >= cum_nblk[None, :]).astype(
            jnp.int32), axis=1), R - 1).astype(jnp.int32)
    seg_start = (blk_off * TB2).astype(jnp.int32)
    expert = ex2.reshape(B)
    slot = jnp.take(seg_start, expert) + pos2.reshape(B)

    z_disp = _sc_dispatch(z, slot)
    nc = _norms(cb_flat)
    zq_disp, xsd = _vq(e_of_blk, z_disp, codebooks, nc.reshape(R, 1, K), T)
    zq_s, xs_s = _sc_undispatch(zq_disp, xsd, slot)
    out, xs = _combine(z, zq_s, xs_s, gate)
    x_sym = xs.reshape(B, KBITS // BPS, 2)
    return (out, x_sym, jp, mi.reshape(B))


# speedup vs baseline: 1.1892x; 1.1892x over previous
import functools

import jax
import jax.numpy as jnp
from jax import lax
from jax.experimental import pallas as pl
from jax.experimental.pallas import tpu as pltpu
from jax.experimental.pallas import tpu_sc as plsc

B = 4096
IN = 128
H = 128
R = 8
MPHY = 4
K = 1024
D = 256
TAU = 1.0
BPS = 2
KBITS = 10
TEMP_MOD = 0.5

TB = 256
NBLK = B // TB
TBR = 512
NBLKR = B // TBR
TB2 = 256
NB2 = B // TB2 + R
NROWS = NB2 * TB2
TH = 64
KC = 128
XSW = 128


def _router_kernel(phi_ref, ln_g_ref, ln_b_ref, W1_ref, b1_ref,
                   W2_ref, b2_ref, We_ref, be_ref, Wm_ref, bm_ref,
                   jp_ref, mi_ref, gate_ref, ex_ref, pos_ref, cnt_ref,
                   run_scr):
    @pl.when(pl.program_id(0) == 0)
    def _():
        run_scr[...] = jnp.zeros((1, R), jnp.float32)

    phi = phi_ref[...]
    mu = jnp.mean(phi, axis=-1, keepdims=True)
    var = jnp.mean((phi - mu) ** 2, axis=-1, keepdims=True)
    phin = (phi - mu) / jnp.sqrt(var + 1e-5) * ln_g_ref[...] + ln_b_ref[...]
    h = jax.nn.gelu(jnp.dot(phin, W1_ref[...]) + b1_ref[...])
    h = jax.nn.gelu(jnp.dot(h, W2_ref[...]) + b2_ref[...])
    logits_e = jnp.dot(h, We_ref[...]) + be_ref[...]
    logits_m = jnp.dot(h, Wm_ref[...]) + bm_ref[...]
    p_e = jax.nn.softmax(logits_e / TAU, axis=-1)
    p_m = jax.nn.softmax(logits_m / TAU, axis=-1)
    jl = jnp.concatenate(
        [logits_e[:, e:e + 1] + logits_m for e in range(R)], axis=1)
    jp = jnp.concatenate(
        [p_e[:, e:e + 1] * p_m for e in range(R)], axis=1)
    iota_j = lax.broadcasted_iota(jnp.int32, (TBR, R * MPHY), 1)
    jl_max = jnp.max(jl, axis=-1, keepdims=True)
    mi = jnp.min(jnp.where(jl == jl_max, iota_j, R * MPHY), axis=-1,
                 keepdims=True)
    gate = jnp.sum(jnp.where(iota_j == mi, jp, 0.0), axis=-1, keepdims=True)
    expert = mi // MPHY

    jp_ref[...] = jp
    mi_ref[...] = mi
    gate_ref[...] = gate
    ex_ref[...] = expert

    iota_e = lax.broadcasted_iota(jnp.int32, (TBR, R), 1)
    oh = (expert == iota_e)
    oh_f = oh.astype(jnp.float32)
    r_i = lax.broadcasted_iota(jnp.int32, (TBR, TBR), 0)
    c_i = lax.broadcasted_iota(jnp.int32, (TBR, TBR), 1)
    ltri = (c_i <= r_i).astype(jnp.float32)
    ranks = jnp.dot(ltri, oh_f)
    cnt_blk = ranks[TBR - 1:TBR, :]
    run_row = run_scr[...]
    pick_rank = jnp.sum(jnp.where(oh, ranks, 0.0), axis=1, keepdims=True)
    pick_run = jnp.sum(jnp.where(oh, run_row, 0.0), axis=1, keepdims=True)
    pos = pick_run + pick_rank - 1.0
    pos_ref[...] = pos.astype(jnp.int32)
    new_run = run_row + cnt_blk
    run_scr[...] = new_run
    cnt_ref[...] = new_run.astype(jnp.int32)


def _router(phi, ln_g, ln_b, W1, b1, W2, b2, We, be, Wm, bm):
    const_spec = lambda shape: pl.BlockSpec(shape, lambda i: (0, 0))
    tok_spec = lambda shape: pl.BlockSpec(shape, lambda i: (i, 0))
    return pl.pallas_call(
        _router_kernel,
        grid=(NBLKR,),
        in_specs=[
            tok_spec((TBR, IN)),
            const_spec((1, IN)),
            const_spec((1, IN)),
            const_spec((IN, H)),
            const_spec((1, H)),
            const_spec((H, H)),
            const_spec((1, H)),
            const_spec((H, R)),
            const_spec((1, R)),
            const_spec((H, MPHY)),
            const_spec((1, MPHY)),
        ],
        out_specs=[
            tok_spec((TBR, R * MPHY)),
            tok_spec((TBR, 1)),
            tok_spec((TBR, 1)),
            tok_spec((TBR, 1)),
            tok_spec((TBR, 1)),
            pl.BlockSpec((1, R), lambda i: (0, 0)),
        ],
        out_shape=[
            jax.ShapeDtypeStruct((B, R * MPHY), jnp.float32),
            jax.ShapeDtypeStruct((B, 1), jnp.int32),
            jax.ShapeDtypeStruct((B, 1), jnp.float32),
            jax.ShapeDtypeStruct((B, 1), jnp.int32),
            jax.ShapeDtypeStruct((B, 1), jnp.int32),
            jax.ShapeDtypeStruct((1, R), jnp.int32),
        ],
        scratch_shapes=[pltpu.VMEM((1, R), jnp.float32)],
    )(phi, ln_g.reshape(1, IN), ln_b.reshape(1, IN), W1, b1.reshape(1, H),
      W2, b2.reshape(1, H), We, be.reshape(1, R), Wm, bm.reshape(1, MPHY))


def _norms_kernel(cb_ref, nc_ref):
    for e in range(R):
        cbe = cb_ref[e * K:(e + 1) * K, :]
        nc_ref[0:1, e * K:(e + 1) * K] = jnp.sum(cbe * cbe, axis=-1)[None, :]


def _norms(cb_flat):
    return pl.pallas_call(
        _norms_kernel,
        out_shape=jax.ShapeDtypeStruct((1, R * K), jnp.float32),
    )(cb_flat)


def _vq_kernel(eb_ref, act_ref, z_ref, cb_ref, nc_ref, T_ref, zq_ref, xs_ref):
  @pl.when(act_ref[pl.program_id(0)] == 1)
  def _active_body():
    z = z_ref[...]
    rn = jnp.sum(z * z, axis=-1, keepdims=True)
    cbe = cb_ref[...].reshape(K, D)
    nce = nc_ref[...].reshape(1, K)
    mm = lax.dot_general(z, cbe, (((1,), (1,)), ((), ())))
    lane = lax.broadcasted_iota(jnp.int32, (TH, KC), 1)
    nh = TB2 // TH
    outs = []
    for hh in range(nh):
        sl = slice(hh * TH, (hh + 1) * TH)
        rnh = rn[sl]
        rmin = None
        for c in range(K // KC):
            mmc = mm[sl, c * KC:(c + 1) * KC]
            ncc = nce[:, c * KC:(c + 1) * KC]
            dm = (rnh - 2.0 * mmc) + ncc
            if c == 0:
                rmin = dm
                ridx = jnp.zeros((TH, KC), jnp.int32)
            else:
                upd = dm < rmin
                ridx = jnp.where(upd, c, ridx)
                rmin = jnp.where(upd, dm, rmin)
        tmin = jnp.min(rmin, axis=-1, keepdims=True)
        fidx = ridx * KC + lane
        isel = jnp.min(jnp.where(rmin == tmin, fidx, K), axis=-1,
                       keepdims=True)
        outs.append(isel)
    code = jnp.concatenate(outs, axis=0)

    iota_full = lax.broadcasted_iota(jnp.int32, (TB2, K), 1)
    ohq = (iota_full == code).astype(jnp.float32)
    zq_ref[...] = jnp.dot(ohq, cbe)

    cols = []
    for s in range(KBITS // BPS):
        pr = jnp.bitwise_and(
            lax.shift_right_logical(code, KBITS - BPS - BPS * s), 3)
        for c in range(2):
            v = jnp.where(pr == 0, T_ref[0, c],
                jnp.where(pr == 1, T_ref[1, c],
                jnp.where(pr == 2, T_ref[2, c], T_ref[3, c])))
            cols.append(v)
    cols.append(jnp.zeros((TB2, XSW - 2 * (KBITS // BPS)), jnp.float32))
    xs_ref[...] = jnp.concatenate(cols, axis=1)


def _vq(e_of_blk, act_blk, z_disp, cb3, nc3, T):
    grid_spec = pltpu.PrefetchScalarGridSpec(
        num_scalar_prefetch=2,
        grid=(NB2,),
        in_specs=[
            pl.BlockSpec((TB2, D), lambda i, eb, act: (i * act[i], 0)),
            pl.BlockSpec((1, K, D), lambda i, eb, act: (eb[i], 0, 0)),
            pl.BlockSpec((1, 1, K), lambda i, eb, act: (eb[i], 0, 0)),
            pl.BlockSpec((4, 2), lambda i, eb, act: (0, 0)),
        ],
        out_specs=[
            pl.BlockSpec((TB2, D), lambda i, eb, act: (i, 0)),
            pl.BlockSpec((TB2, XSW), lambda i, eb, act: (i, 0)),
        ],
    )
    return pl.pallas_call(
        _vq_kernel,
        grid_spec=grid_spec,
        out_shape=[
            jax.ShapeDtypeStruct((NROWS, D), jnp.float32),
            jax.ShapeDtypeStruct((NROWS, XSW), jnp.float32),
        ],
    )(e_of_blk, act_blk, z_disp, cb3, nc3, T)


def _sc_info():
    info = plsc.get_sparse_core_info()
    return info.num_cores, info.num_subcores


def _sc_dispatch(z, slot):
    nc, ns = _sc_info()
    nw = nc * ns
    bpw = B // nw
    mesh = plsc.VectorSubcoreMesh(core_axis_name="c", subcore_axis_name="s")

    @functools.partial(
        pl.kernel,
        out_type=jax.ShapeDtypeStruct((NROWS, D), jnp.float32),
        mesh=mesh,
        scratch_types=[
            pltpu.VMEM((bpw,), jnp.int32),
            pltpu.VMEM((bpw, D), jnp.float32),
            pltpu.SemaphoreType.DMA,
        ],
    )
    def disp_k(z_hbm, slot_hbm, out_hbm, slot_v, rows_v, sem):
        wid = lax.axis_index("s") * nc + lax.axis_index("c")
        base = wid * bpw
        pltpu.sync_copy(slot_hbm.at[pl.ds(base, bpw)], slot_v)
        pltpu.sync_copy(z_hbm.at[pl.ds(base, bpw)], rows_v)
        pltpu.async_copy(rows_v, out_hbm.at[slot_v], sem).wait()

    return disp_k(z, slot)


def _sc_undispatch(zq_disp, xsd, slot):
    nc, ns = _sc_info()
    nw = nc * ns
    bpw = B // nw
    mesh = plsc.VectorSubcoreMesh(core_axis_name="c", subcore_axis_name="s")

    @functools.partial(
        pl.kernel,
        out_type=[
            jax.ShapeDtypeStruct((B, D), jnp.float32),
            jax.ShapeDtypeStruct((B, XSW), jnp.float32),
        ],
        mesh=mesh,
        scratch_types=[
            pltpu.VMEM((bpw,), jnp.int32),
            pltpu.VMEM((bpw, D), jnp.float32),
            pltpu.VMEM((bpw, XSW), jnp.float32),
            pltpu.SemaphoreType.DMA,
            pltpu.SemaphoreType.DMA,
        ],
    )
    def und_k(zqd_hbm, xsd_hbm, slot_hbm, zq_hbm, xss_hbm,
              slot_v, rows_v, xsr_v, sem, sem2):
        wid = lax.axis_index("s") * nc + lax.axis_index("c")
        base = wid * bpw
        pltpu.sync_copy(slot_hbm.at[pl.ds(base, bpw)], slot_v)
        a = pltpu.async_copy(zqd_hbm.at[slot_v], rows_v, sem)
        b = pltpu.async_copy(xsd_hbm.at[slot_v], xsr_v, sem2)
        a.wait()
        pltpu.sync_copy(rows_v, zq_hbm.at[pl.ds(base, bpw)])
        b.wait()
        pltpu.sync_copy(xsr_v, xss_hbm.at[pl.ds(base, bpw)])

    return und_k(zq_disp, xsd, slot)


def _combine_kernel(z_ref, zq_ref, xss_ref, gate_ref, out_ref, xs_ref):
    z = z_ref[...]
    zq = zq_ref[...]
    out_ref[...] = (z + (zq - z)) * gate_ref[...]
    xs_ref[...] = xss_ref[:, 0:2 * (KBITS // BPS)]


def _combine(z, zq_s, xs_s, gate):
    tok_spec = lambda shape: pl.BlockSpec(shape, lambda i: (i, 0))
    return pl.pallas_call(
        _combine_kernel,
        grid=(NBLK,),
        in_specs=[
            tok_spec((TB, D)),
            tok_spec((TB, D)),
            tok_spec((TB, XSW)),
            tok_spec((TB, 1)),
        ],
        out_specs=[tok_spec((TB, D)), tok_spec((TB, 2 * (KBITS // BPS)))],
        out_shape=[
            jax.ShapeDtypeStruct((B, D), jnp.float32),
            jax.ShapeDtypeStruct((B, 2 * (KBITS // BPS)), jnp.float32),
        ],
    )(z, zq_s, xs_s, gate)


def _int_to_bits(x, num_bits):
    shifts = jnp.arange(num_bits - 1, -1, -1)
    return ((x[..., None] >> shifts) & 1).astype(jnp.float32)


def _qam_table():
    import numpy as np
    m_side = int(np.sqrt(1 << BPS))
    levels = jnp.arange(-(m_side - 1), m_side + 1, 2).astype(jnp.float32)
    xs, ys = jnp.meshgrid(levels, levels, indexing='ij')
    pts = jnp.stack([xs.reshape(-1), ys.reshape(-1)], axis=-1)
    max_power = (pts ** 2).sum(axis=-1).max()
    const = pts / jnp.sqrt(max_power + 1e-9)
    cand_bits = _int_to_bits(jnp.arange(1 << BPS), BPS)
    patt = cand_bits
    d_bits = ((patt[:, None, :] - cand_bits[None, :, :]) ** 2).sum(axis=-1)
    w_sym = jax.nn.softmax(-d_bits / max(TEMP_MOD, 1e-6), axis=1)
    return w_sym @ const


def kernel(z, phi, ln_g, ln_b, W1, b1, W2, b2, We, be, Wm, bm, codebooks):
    cb_flat = codebooks.reshape(R * K, D)
    T = _qam_table()

    jp, mi, gate, ex2, pos2, cnt2 = _router(phi, ln_g, ln_b, W1, b1, W2, b2,
                                            We, be, Wm, bm)

    counts = cnt2.reshape(R)
    nblk_e = (counts + TB2 - 1) // TB2
    cum_nblk = jnp.cumsum(nblk_e)
    blk_off = cum_nblk - nblk_e
    e_of_blk = jnp.minimum(
        jnp.sum((jnp.arange(NB2)[:, None] >= cum_nblk[None, :]).astype(
            jnp.int32), axis=1), R - 1).astype(jnp.int32)
    seg_start = (blk_off * TB2).astype(jnp.int32)
    act_blk = (jnp.arange(NB2) < cum_nblk[R - 1]).astype(jnp.int32)
    expert = ex2.reshape(B)
    slot = jnp.take(seg_start, expert) + pos2.reshape(B)

    z_disp = _sc_dispatch(z, slot)
    nc = _norms(cb_flat)
    zq_disp, xsd = _vq(e_of_blk, act_blk, z_disp, codebooks,
                       nc.reshape(R, 1, K), T)
    zq_s, xs_s = _sc_undispatch(zq_disp, xsd, slot)
    out, xs = _combine(z, zq_s, xs_s, gate)
    x_sym = xs.reshape(B, KBITS // BPS, 2)
    return (out, x_sym, jp, mi.reshape(B))


# speedup vs baseline: 1.2712x; 1.0689x over previous
import functools

import jax
import jax.numpy as jnp
from jax import lax
from jax.experimental import pallas as pl
from jax.experimental.pallas import tpu as pltpu
from jax.experimental.pallas import tpu_sc as plsc

B = 4096
IN = 128
H = 128
R = 8
MPHY = 4
K = 1024
D = 256
TAU = 1.0
BPS = 2
KBITS = 10
TEMP_MOD = 0.5

TB = 256
NBLK = B // TB
TBR = 1024
NBLKR = B // TBR
TB2 = 256
NB2 = B // TB2 + R
NROWS = NB2 * TB2
TH = 64
KC = 128
XSW = 128


def _router_kernel(phi_ref, ln_g_ref, ln_b_ref, W1_ref, b1_ref,
                   W2_ref, b2_ref, We_ref, be_ref, Wm_ref, bm_ref,
                   jp_ref, mi_ref, gate_ref, ex_ref, pos_ref, cnt_ref,
                   run_scr):
    @pl.when(pl.program_id(0) == 0)
    def _():
        run_scr[...] = jnp.zeros((1, R), jnp.float32)

    phi = phi_ref[...]
    mu = jnp.mean(phi, axis=-1, keepdims=True)
    var = jnp.mean((phi - mu) ** 2, axis=-1, keepdims=True)
    phin = (phi - mu) / jnp.sqrt(var + 1e-5) * ln_g_ref[...] + ln_b_ref[...]
    h = jax.nn.gelu(jnp.dot(phin, W1_ref[...]) + b1_ref[...])
    h = jax.nn.gelu(jnp.dot(h, W2_ref[...]) + b2_ref[...])
    logits_e = jnp.dot(h, We_ref[...]) + be_ref[...]
    logits_m = jnp.dot(h, Wm_ref[...]) + bm_ref[...]
    p_e = jax.nn.softmax(logits_e / TAU, axis=-1)
    p_m = jax.nn.softmax(logits_m / TAU, axis=-1)
    jl = jnp.concatenate(
        [logits_e[:, e:e + 1] + logits_m for e in range(R)], axis=1)
    jp = jnp.concatenate(
        [p_e[:, e:e + 1] * p_m for e in range(R)], axis=1)
    iota_j = lax.broadcasted_iota(jnp.int32, (TBR, R * MPHY), 1)
    jl_max = jnp.max(jl, axis=-1, keepdims=True)
    mi = jnp.min(jnp.where(jl == jl_max, iota_j, R * MPHY), axis=-1,
                 keepdims=True)
    gate = jnp.sum(jnp.where(iota_j == mi, jp, 0.0), axis=-1, keepdims=True)
    expert = mi // MPHY

    jp_ref[...] = jp
    mi_ref[...] = mi
    gate_ref[...] = gate
    ex_ref[...] = expert

    iota_e = lax.broadcasted_iota(jnp.int32, (TBR, R), 1)
    oh = (expert == iota_e)
    oh_f = oh.astype(jnp.float32)
    r_i = lax.broadcasted_iota(jnp.int32, (TBR, TBR), 0)
    c_i = lax.broadcasted_iota(jnp.int32, (TBR, TBR), 1)
    ltri = (c_i <= r_i).astype(jnp.float32)
    ranks = jnp.dot(ltri, oh_f)
    cnt_blk = ranks[TBR - 1:TBR, :]
    run_row = run_scr[...]
    pick_rank = jnp.sum(jnp.where(oh, ranks, 0.0), axis=1, keepdims=True)
    pick_run = jnp.sum(jnp.where(oh, run_row, 0.0), axis=1, keepdims=True)
    pos = pick_run + pick_rank - 1.0
    pos_ref[...] = pos.astype(jnp.int32)
    new_run = run_row + cnt_blk
    run_scr[...] = new_run
    cnt_ref[...] = new_run.astype(jnp.int32)


def _router(phi, ln_g, ln_b, W1, b1, W2, b2, We, be, Wm, bm):
    const_spec = lambda shape: pl.BlockSpec(shape, lambda i: (0, 0))
    tok_spec = lambda shape: pl.BlockSpec(shape, lambda i: (i, 0))
    return pl.pallas_call(
        _router_kernel,
        grid=(NBLKR,),
        in_specs=[
            tok_spec((TBR, IN)),
            const_spec((1, IN)),
            const_spec((1, IN)),
            const_spec((IN, H)),
            const_spec((1, H)),
            const_spec((H, H)),
            const_spec((1, H)),
            const_spec((H, R)),
            const_spec((1, R)),
            const_spec((H, MPHY)),
            const_spec((1, MPHY)),
        ],
        out_specs=[
            tok_spec((TBR, R * MPHY)),
            tok_spec((TBR, 1)),
            tok_spec((TBR, 1)),
            tok_spec((TBR, 1)),
            tok_spec((TBR, 1)),
            pl.BlockSpec((1, R), lambda i: (0, 0)),
        ],
        out_shape=[
            jax.ShapeDtypeStruct((B, R * MPHY), jnp.float32),
            jax.ShapeDtypeStruct((B, 1), jnp.int32),
            jax.ShapeDtypeStruct((B, 1), jnp.float32),
            jax.ShapeDtypeStruct((B, 1), jnp.int32),
            jax.ShapeDtypeStruct((B, 1), jnp.int32),
            jax.ShapeDtypeStruct((1, R), jnp.int32),
        ],
        scratch_shapes=[pltpu.VMEM((1, R), jnp.float32)],
    )(phi, ln_g.reshape(1, IN), ln_b.reshape(1, IN), W1, b1.reshape(1, H),
      W2, b2.reshape(1, H), We, be.reshape(1, R), Wm, bm.reshape(1, MPHY))


def _norms_kernel(cb_ref, nc_ref):
    for e in range(R):
        cbe = cb_ref[e * K:(e + 1) * K, :]
        nc_ref[0:1, e * K:(e + 1) * K] = jnp.sum(cbe * cbe, axis=-1)[None, :]


def _norms(cb_flat):
    return pl.pallas_call(
        _norms_kernel,
        out_shape=jax.ShapeDtypeStruct((1, R * K), jnp.float32),
    )(cb_flat)


def _vq_kernel(eb_ref, act_ref, z_ref, cb_ref, nc_ref, T_ref, zq_ref, xs_ref):
  @pl.when(act_ref[pl.program_id(0)] == 1)
  def _active_body():
    z = z_ref[...]
    rn = jnp.sum(z * z, axis=-1, keepdims=True)
    cbe = cb_ref[...].reshape(K, D)
    nce = nc_ref[...].reshape(1, K)
    mm = lax.dot_general(z, cbe, (((1,), (1,)), ((), ())))
    lane = lax.broadcasted_iota(jnp.int32, (TH, KC), 1)
    nh = TB2 // TH
    outs = []
    for hh in range(nh):
        sl = slice(hh * TH, (hh + 1) * TH)
        rnh = rn[sl]
        rmin = None
        for c in range(K // KC):
            mmc = mm[sl, c * KC:(c + 1) * KC]
            ncc = nce[:, c * KC:(c + 1) * KC]
            dm = (rnh - 2.0 * mmc) + ncc
            if c == 0:
                rmin = dm
                ridx = jnp.zeros((TH, KC), jnp.int32)
            else:
                upd = dm < rmin
                ridx = jnp.where(upd, c, ridx)
                rmin = jnp.where(upd, dm, rmin)
        tmin = jnp.min(rmin, axis=-1, keepdims=True)
        fidx = ridx * KC + lane
        isel = jnp.min(jnp.where(rmin == tmin, fidx, K), axis=-1,
                       keepdims=True)
        outs.append(isel)
    code = jnp.concatenate(outs, axis=0)

    iota_full = lax.broadcasted_iota(jnp.int32, (TB2, K), 1)
    ohq = (iota_full == code).astype(jnp.float32)
    zq_ref[...] = jnp.dot(ohq, cbe)

    cols = []
    for s in range(KBITS // BPS):
        pr = jnp.bitwise_and(
            lax.shift_right_logical(code, KBITS - BPS - BPS * s), 3)
        for c in range(2):
            v = jnp.where(pr == 0, T_ref[0, c],
                jnp.where(pr == 1, T_ref[1, c],
                jnp.where(pr == 2, T_ref[2, c], T_ref[3, c])))
            cols.append(v)
    cols.append(jnp.zeros((TB2, XSW - 2 * (KBITS // BPS)), jnp.float32))
    xs_ref[...] = jnp.concatenate(cols, axis=1)


def _vq(e_of_blk, act_blk, z_disp, cb3, nc3, T):
    grid_spec = pltpu.PrefetchScalarGridSpec(
        num_scalar_prefetch=2,
        grid=(NB2,),
        in_specs=[
            pl.BlockSpec((TB2, D), lambda i, eb, act: (i * act[i], 0)),
            pl.BlockSpec((1, K, D), lambda i, eb, act: (eb[i], 0, 0)),
            pl.BlockSpec((1, 1, K), lambda i, eb, act: (eb[i], 0, 0)),
            pl.BlockSpec((4, 2), lambda i, eb, act: (0, 0)),
        ],
        out_specs=[
            pl.BlockSpec((TB2, D), lambda i, eb, act: (i, 0)),
            pl.BlockSpec((TB2, XSW), lambda i, eb, act: (i, 0)),
        ],
    )
    return pl.pallas_call(
        _vq_kernel,
        grid_spec=grid_spec,
        out_shape=[
            jax.ShapeDtypeStruct((NROWS, D), jnp.float32),
            jax.ShapeDtypeStruct((NROWS, XSW), jnp.float32),
        ],
    )(e_of_blk, act_blk, z_disp, cb3, nc3, T)


def _sc_info():
    info = plsc.get_sparse_core_info()
    return info.num_cores, info.num_subcores


def _sc_dispatch(z, slot):
    nc, ns = _sc_info()
    nw = nc * ns
    bpw = B // nw
    mesh = plsc.VectorSubcoreMesh(core_axis_name="c", subcore_axis_name="s")

    @functools.partial(
        pl.kernel,
        out_type=jax.ShapeDtypeStruct((NROWS, D), jnp.float32),
        mesh=mesh,
        scratch_types=[
            pltpu.VMEM((bpw,), jnp.int32),
            pltpu.VMEM((bpw, D), jnp.float32),
            pltpu.SemaphoreType.DMA,
        ],
    )
    def disp_k(z_hbm, slot_hbm, out_hbm, slot_v, rows_v, sem):
        wid = lax.axis_index("s") * nc + lax.axis_index("c")
        base = wid * bpw
        pltpu.sync_copy(slot_hbm.at[pl.ds(base, bpw)], slot_v)
        pltpu.sync_copy(z_hbm.at[pl.ds(base, bpw)], rows_v)
        pltpu.async_copy(rows_v, out_hbm.at[slot_v], sem).wait()

    return disp_k(z, slot)


def _sc_undispatch(zq_disp, xsd, slot):
    nc, ns = _sc_info()
    nw = nc * ns
    bpw = B // nw
    mesh = plsc.VectorSubcoreMesh(core_axis_name="c", subcore_axis_name="s")

    @functools.partial(
        pl.kernel,
        out_type=[
            jax.ShapeDtypeStruct((B, D), jnp.float32),
            jax.ShapeDtypeStruct((B, XSW), jnp.float32),
        ],
        mesh=mesh,
        scratch_types=[
            pltpu.VMEM((bpw,), jnp.int32),
            pltpu.VMEM((bpw, D), jnp.float32),
            pltpu.VMEM((bpw, XSW), jnp.float32),
            pltpu.SemaphoreType.DMA,
            pltpu.SemaphoreType.DMA,
        ],
    )
    def und_k(zqd_hbm, xsd_hbm, slot_hbm, zq_hbm, xss_hbm,
              slot_v, rows_v, xsr_v, sem, sem2):
        wid = lax.axis_index("s") * nc + lax.axis_index("c")
        base = wid * bpw
        pltpu.sync_copy(slot_hbm.at[pl.ds(base, bpw)], slot_v)
        a = pltpu.async_copy(zqd_hbm.at[slot_v], rows_v, sem)
        b = pltpu.async_copy(xsd_hbm.at[slot_v], xsr_v, sem2)
        a.wait()
        pltpu.sync_copy(rows_v, zq_hbm.at[pl.ds(base, bpw)])
        b.wait()
        pltpu.sync_copy(xsr_v, xss_hbm.at[pl.ds(base, bpw)])

    return und_k(zq_disp, xsd, slot)


def _combine_kernel(z_ref, zq_ref, xss_ref, gate_ref, out_ref, xs_ref):
    z = z_ref[...]
    zq = zq_ref[...]
    out_ref[...] = (z + (zq - z)) * gate_ref[...]
    xs_ref[...] = xss_ref[:, 0:2 * (KBITS // BPS)]


def _combine(z, zq_s, xs_s, gate):
    tok_spec = lambda shape: pl.BlockSpec(shape, lambda i: (i, 0))
    return pl.pallas_call(
        _combine_kernel,
        grid=(NBLK,),
        in_specs=[
            tok_spec((TB, D)),
            tok_spec((TB, D)),
            tok_spec((TB, XSW)),
            tok_spec((TB, 1)),
        ],
        out_specs=[tok_spec((TB, D)), tok_spec((TB, 2 * (KBITS // BPS)))],
        out_shape=[
            jax.ShapeDtypeStruct((B, D), jnp.float32),
            jax.ShapeDtypeStruct((B, 2 * (KBITS // BPS)), jnp.float32),
        ],
    )(z, zq_s, xs_s, gate)


def _int_to_bits(x, num_bits):
    shifts = jnp.arange(num_bits - 1, -1, -1)
    return ((x[..., None] >> shifts) & 1).astype(jnp.float32)


def _qam_table():
    import numpy as np
    m_side = int(np.sqrt(1 << BPS))
    levels = jnp.arange(-(m_side - 1), m_side + 1, 2).astype(jnp.float32)
    xs, ys = jnp.meshgrid(levels, levels, indexing='ij')
    pts = jnp.stack([xs.reshape(-1), ys.reshape(-1)], axis=-1)
    max_power = (pts ** 2).sum(axis=-1).max()
    const = pts / jnp.sqrt(max_power + 1e-9)
    cand_bits = _int_to_bits(jnp.arange(1 << BPS), BPS)
    patt = cand_bits
    d_bits = ((patt[:, None, :] - cand_bits[None, :, :]) ** 2).sum(axis=-1)
    w_sym = jax.nn.softmax(-d_bits / max(TEMP_MOD, 1e-6), axis=1)
    return w_sym @ const


def kernel(z, phi, ln_g, ln_b, W1, b1, W2, b2, We, be, Wm, bm, codebooks):
    cb_flat = codebooks.reshape(R * K, D)
    T = _qam_table()

    jp, mi, gate, ex2, pos2, cnt2 = _router(phi, ln_g, ln_b, W1, b1, W2, b2,
                                            We, be, Wm, bm)

    counts = cnt2.reshape(R)
    nblk_e = (counts + TB2 - 1) // TB2
    cum_nblk = jnp.cumsum(nblk_e)
    blk_off = cum_nblk - nblk_e
    e_of_blk = jnp.minimum(
        jnp.sum((jnp.arange(NB2)[:, None] >= cum_nblk[None, :]).astype(
            jnp.int32), axis=1), R - 1).astype(jnp.int32)
    seg_start = (blk_off * TB2).astype(jnp.int32)
    act_blk = (jnp.arange(NB2) < cum_nblk[R - 1]).astype(jnp.int32)
    expert = ex2.reshape(B)
    slot = jnp.take(seg_start, expert) + pos2.reshape(B)

    z_disp = _sc_dispatch(z, slot)
    nc = _norms(cb_flat)
    zq_disp, xsd = _vq(e_of_blk, act_blk, z_disp, codebooks,
                       nc.reshape(R, 1, K), T)
    zq_s, xs_s = _sc_undispatch(zq_disp, xsd, slot)
    out, xs = _combine(z, zq_s, xs_s, gate)
    x_sym = xs.reshape(B, KBITS // BPS, 2)
    return (out, x_sym, jp, mi.reshape(B))
